# baseline (speedup 1.0000x reference)
import functools

import jax
import jax.numpy as jnp
from jax import lax
from jax.experimental import pallas as pl
from jax.experimental.pallas import tpu as pltpu
from jax.experimental.pallas import tpu_sc as plsc

F32 = jnp.float32
BF16 = jnp.bfloat16

_NC = 2
_NS = 16
_NW = _NC * _NS
_CH = 80

_SC_PARAMS = pltpu.CompilerParams(use_tc_tiling_on_sc=False,
                                  needs_layout_passes=False)


def _mesh():
    return plsc.VectorSubcoreMesh(core_axis_name="c", subcore_axis_name="s",
                                  num_cores=_NC, num_subcores=_NS)


def _rpt(n):
    return (-(-n // _NS) + 7) // 8 * 8


def _prep_tc(node_attr, wd, ws, n_blk=1000):
    n, nd = node_attr.shape
    h = wd.shape[1]

    def body(na_ref, wd_ref, ws_ref, a_ref, b_ref):
        na = na_ref[...]
        a_ref[...] = jnp.dot(na, wd_ref[...],
                             preferred_element_type=F32).astype(BF16)
        b_ref[...] = jnp.dot(na, ws_ref[...],
                             preferred_element_type=F32).astype(BF16)

    return pl.pallas_call(
        body,
        grid=(n // n_blk,),
        in_specs=[
            pl.BlockSpec((n_blk, nd), lambda i: (i, 0)),
            pl.BlockSpec((nd, h), lambda i: (0, 0)),
            pl.BlockSpec((nd, h), lambda i: (0, 0)),
        ],
        out_specs=[pl.BlockSpec((n_blk, h), lambda i: (i, 0))] * 2,
        out_shape=[jax.ShapeDtypeStruct((n, h), BF16)] * 2,
    )(node_attr, wd, ws)


def _sc_gather_sum(a, b, src2d, dst2d):
    n, h = a.shape
    nrow = src2d.shape[0]
    nit = nrow // _NW
    hw = h // 32
    rpt = _rpt(n)
    last = n - rpt * (_NS - 1)

    @functools.partial(
        pl.kernel,
        out_type=jax.ShapeDtypeStruct((nrow * _CH, h), BF16),
        mesh=_mesh(),
        compiler_params=_SC_PARAMS,
        scratch_types=[
            pltpu.VMEM_SHARED((n, h), BF16),
            pltpu.VMEM_SHARED((n, h), BF16),
            pltpu.VMEM((nit, _CH), jnp.int32),
            pltpu.VMEM((2, _CH), jnp.int32),
            pltpu.VMEM((2, _CH, h), BF16),
            pltpu.VMEM((2, _CH, h), BF16),
            pltpu.VMEM((2, _CH, h), BF16),
            pltpu.SemaphoreType.DMA,
            pltpu.SemaphoreType.DMA,
            pltpu.SemaphoreType.DMA,
            pltpu.SemaphoreType.DMA,
            pltpu.SemaphoreType.DMA,
            pltpu.SemaphoreType.DMA,
            pltpu.SemaphoreType.DMA,
            pltpu.SemaphoreType.DMA,
        ],
    )
    def k(a_hbm, b_hbm, s2_hbm, d2_hbm, out_hbm, asp, bsp, srcv, dsv,
          ra, rb, ro, sa0, sa1, sb0, sb1, sw0, sw1, si0, si1):
        cid = lax.axis_index("c")
        sid = lax.axis_index("s")
        wid = sid * _NC + cid
        row0 = wid * nit
        e0 = row0 * _CH

        @pl.when(sid < _NS - 1)
        def _():
            pltpu.sync_copy(a_hbm.at[pl.ds(sid * rpt, rpt)],
                            asp.at[pl.ds(sid * rpt, rpt)])
            pltpu.sync_copy(b_hbm.at[pl.ds(sid * rpt, rpt)],
                            bsp.at[pl.ds(sid * rpt, rpt)])

        @pl.when(sid == _NS - 1)
        def _():
            pltpu.sync_copy(a_hbm.at[pl.ds(rpt * (_NS - 1), last)],
                            asp.at[pl.ds(rpt * (_NS - 1), last)])
            pltpu.sync_copy(b_hbm.at[pl.ds(rpt * (_NS - 1), last)],
                            bsp.at[pl.ds(rpt * (_NS - 1), last)])

        pltpu.sync_copy(s2_hbm.at[pl.ds(row0, nit)], srcv)
        for bs in range(2):
            pltpu.sync_copy(d2_hbm.at[row0 + bs], dsv.at[bs])
        plsc.subcore_barrier()
        sa = (sa0, sa1)
        sb = (sb0, sb1)
        sw = (sw0, sw1)
        si = (si0, si1)
        for bs in range(2):
            pltpu.async_copy(asp.at[dsv.at[bs]], ra.at[bs], sa[bs])
            pltpu.async_copy(bsp.at[srcv.at[bs]], rb.at[bs], sb[bs])

        def pair(g, carry):
            for bs in range(2):
                c = 2 * g + bs
                pltpu.make_async_copy(asp.at[dsv.at[bs]], ra.at[bs],
                                      sa[bs]).wait()
                pltpu.make_async_copy(bsp.at[srcv.at[c]], rb.at[bs],
                                      sb[bs]).wait()

                @pl.when(c + 2 < nit)
                def _():
                    pltpu.async_copy(d2_hbm.at[row0 + c + 2], dsv.at[bs],
                                     si[bs])

                @pl.when(c >= 2)
                def _():
                    pltpu.make_async_copy(
                        ro.at[bs], out_hbm.at[pl.ds(e0 + (c - 2) * _CH, _CH)],
                        sw[bs]).wait()

                def rowf(r, c2):
                    for j in range(hw):
                        sl = pl.ds(j * 32, 32)
                        ro[bs, r, sl] = ra[bs, r, sl] + rb[bs, r, sl]
                    return c2

                lax.fori_loop(0, _CH, rowf, 0, unroll=2)
                pltpu.async_copy(ro.at[bs],
                                 out_hbm.at[pl.ds(e0 + c * _CH, _CH)], sw[bs])

                @pl.when(c + 2 < nit)
                def _():
                    pltpu.make_async_copy(d2_hbm.at[row0 + c + 2],
                                          dsv.at[bs], si[bs]).wait()
                    pltpu.async_copy(asp.at[dsv.at[bs]], ra.at[bs], sa[bs])
                    pltpu.async_copy(bsp.at[srcv.at[c + 2]], rb.at[bs],
                                     sb[bs])
            return carry

        lax.fori_loop(0, nit // 2, pair, 0)
        for bs in range(2):
            c = nit - 2 + bs
            pltpu.make_async_copy(ro.at[bs],
                                  out_hbm.at[pl.ds(e0 + c * _CH, _CH)],
                                  sw[bs]).wait()

    return k(a, b, src2d, dst2d)


def _edge_mlp_tc(hpre, ea, w1e, b1, w2, b2, nreal, ea_off, e_blk=2000):
    e, h = hpre.shape
    ed = ea.shape[1]
    off = ea_off // e_blk

    def body(hp_ref, ea_ref, w1_ref, b1_ref, w2_ref, b2_ref, out_ref):
        hh = hp_ref[...].astype(F32) + jnp.dot(
            ea_ref[...], w1_ref[...],
            preferred_element_type=F32) + b1_ref[...]
        hh = jnp.maximum(hh, 0.0)
        out_ref[...] = jnp.dot(hh, w2_ref[...],
                               preferred_element_type=F32) + b2_ref[...]

    return pl.pallas_call(
        body,
        grid=(nreal // e_blk,),
        in_specs=[
            pl.BlockSpec((e_blk, h), lambda i: (i, 0)),
            pl.BlockSpec((e_blk, ed), lambda i: (i + off, 0)),
            pl.BlockSpec((ed, h), lambda i: (0, 0)),
            pl.BlockSpec((1, h), lambda i: (0, 0)),
            pl.BlockSpec((h, h), lambda i: (0, 0)),
            pl.BlockSpec((1, h), lambda i: (0, 0)),
        ],
        out_specs=pl.BlockSpec((e_blk, h), lambda i: (i, 0)),
        out_shape=jax.ShapeDtypeStruct((e, h), F32),
    )(hpre, ea, w1e, b1, w2, b2)


def _sc_scatter_add(e_upd, dst2d, zrows, n):
    e, h = e_upd.shape
    nrow = dst2d.shape[0]
    nit = nrow // _NW
    rpt = _rpt(n + 1)
    npad = rpt * _NS

    @functools.partial(
        pl.kernel,
        out_type=jax.ShapeDtypeStruct((_NC, npad, h), F32),
        mesh=_mesh(),
        compiler_params=_SC_PARAMS,
        scratch_types=[
            pltpu.VMEM_SHARED((npad, h), F32),
            pltpu.VMEM((nit, _CH), jnp.int32),
            pltpu.VMEM((2, _CH, h), F32),
            pltpu.SemaphoreType.DMA,
            pltpu.SemaphoreType.DMA,
            pltpu.SemaphoreType.DMA,
            pltpu.SemaphoreType.DMA,
        ],
    )
    def k(eu_hbm, d2_hbm, z_hbm, out_hbm, acc, dstv, buf, sr0, sr1, ss0, ss1):
        cid = lax.axis_index("c")
        sid = lax.axis_index("s")
        wid = sid * _NC + cid
        row0 = wid * nit
        e0 = row0 * _CH
        pltpu.sync_copy(d2_hbm.at[pl.ds(row0, nit)], dstv)
        pltpu.sync_copy(z_hbm, acc.at[pl.ds(sid * rpt, rpt)])
        plsc.subcore_barrier()
        sr = (sr0, sr1)
        ss = (ss0, ss1)
        for bs in range(2):
            pltpu.async_copy(eu_hbm.at[pl.ds(e0 + bs * _CH, _CH)], buf.at[bs],
                             sr[bs])

        def pair(g, carry):
            for bs in range(2):
                c = 2 * g + bs
                pltpu.make_async_copy(eu_hbm.at[pl.ds(e0 + c * _CH, _CH)],
                                      buf.at[bs], sr[bs]).wait()
                pltpu.async_copy(buf.at[bs], acc.at[dstv.at[c]], ss[bs],
                                 add=True)
                pltpu.make_async_copy(buf.at[bs], acc.at[dstv.at[c]],
                                      ss[bs]).wait()

                @pl.when(c + 2 < nit)
                def _():
                    pltpu.async_copy(eu_hbm.at[pl.ds(e0 + (c + 2) * _CH, _CH)],
                                     buf.at[bs], sr[bs])
            return carry

        lax.fori_loop(0, nit // 2, pair, 0)
        plsc.subcore_barrier()
        pltpu.sync_copy(acc.at[pl.ds(sid * rpt, rpt)],
                        out_hbm.at[cid, pl.ds(sid * rpt, rpt)])

    return k(e_upd, dst2d, zrows)


def _node_mlp_tc(na, a0, a1, w1a, w1b, b1, w2, b2, n_blk=1000):
    n, nd = na.shape
    h = w1a.shape[1]

    def body(na_ref, a0_ref, a1_ref, w1a_ref, w1b_ref, b1_ref, w2_ref, b2_ref,
             out_ref, obf_ref):
        agg = a0_ref[...] + a1_ref[...]
        hh = jnp.dot(na_ref[...], w1a_ref[...], preferred_element_type=F32)
        hh = hh + jnp.dot(agg, w1b_ref[...], preferred_element_type=F32)
        hh = jnp.maximum(hh + b1_ref[...], 0.0)
        nu = jnp.dot(hh, w2_ref[...], preferred_element_type=F32) + b2_ref[...]
        out_ref[...] = nu
        obf_ref[...] = nu.astype(BF16)

    return pl.pallas_call(
        body,
        grid=(n // n_blk,),
        in_specs=[
            pl.BlockSpec((n_blk, nd), lambda i: (i, 0)),
            pl.BlockSpec((n_blk, h), lambda i: (i, 0)),
            pl.BlockSpec((n_blk, h), lambda i: (i, 0)),
            pl.BlockSpec((nd, h), lambda i: (0, 0)),
            pl.BlockSpec((h, h), lambda i: (0, 0)),
            pl.BlockSpec((1, h), lambda i: (0, 0)),
            pl.BlockSpec((h, h), lambda i: (0, 0)),
            pl.BlockSpec((1, h), lambda i: (0, 0)),
        ],
        out_specs=[pl.BlockSpec((n_blk, h), lambda i: (i, 0))] * 2,
        out_shape=[jax.ShapeDtypeStruct((n, h), F32),
                   jax.ShapeDtypeStruct((n, h), BF16)],
    )(na, a0, a1, w1a, w1b, b1, w2, b2)


def _sc_nbr_scatter(ext, src2d, dsts2d, zrows, n):
    hx = ext.shape[1]
    nrow = src2d.shape[0]
    nit = nrow // _NW
    rpt = _rpt(n + 1)
    npad = rpt * _NS

    @functools.partial(
        pl.kernel,
        out_type=jax.ShapeDtypeStruct((_NC, npad, hx), F32),
        mesh=_mesh(),
        compiler_params=_SC_PARAMS,
        scratch_types=[
            pltpu.VMEM_SHARED((npad, hx), F32),
            pltpu.VMEM((nit, _CH), jnp.int32),
            pltpu.VMEM((2, _CH), jnp.int32),
            pltpu.VMEM((2, _CH, hx), F32),
            pltpu.SemaphoreType.DMA,
            pltpu.SemaphoreType.DMA,
            pltpu.SemaphoreType.DMA,
            pltpu.SemaphoreType.DMA,
            pltpu.SemaphoreType.DMA,
            pltpu.SemaphoreType.DMA,
        ],
    )
    def k(ext_hbm, s2_hbm, ds2_hbm, z_hbm, nbr_hbm,
          acc, srcv, dsv, rs, sg0, sg1, si0, si1, sc0, sc1):
        cid = lax.axis_index("c")
        sid = lax.axis_index("s")
        wid = sid * _NC + cid
        row0 = wid * nit
        pltpu.sync_copy(s2_hbm.at[pl.ds(row0, nit)], srcv)
        pltpu.sync_copy(z_hbm, acc.at[pl.ds(sid * rpt, rpt)])
        plsc.subcore_barrier()
        gsem = (sg0, sg1)
        isem = (si0, si1)
        csem = (sc0, sc1)
        for bs in range(2):
            pltpu.async_copy(ext_hbm.at[srcv.at[bs]], rs.at[bs], gsem[bs])
            pltpu.async_copy(ds2_hbm.at[row0 + bs], dsv.at[bs], isem[bs])

        def pair(g, carry):
            for bs in range(2):
                c = 2 * g + bs
                pltpu.make_async_copy(ext_hbm.at[srcv.at[c]], rs.at[bs],
                                      gsem[bs]).wait()
                pltpu.make_async_copy(ds2_hbm.at[row0 + c], dsv.at[bs],
                                      isem[bs]).wait()
                pltpu.async_copy(rs.at[bs], acc.at[dsv.at[bs]], csem[bs],
                                 add=True)
                pltpu.make_async_copy(rs.at[bs], acc.at[dsv.at[bs]],
                                      csem[bs]).wait()

                @pl.when(c + 2 < nit)
                def _():
                    pltpu.async_copy(ext_hbm.at[srcv.at[c + 2]], rs.at[bs],
                                     gsem[bs])
                    pltpu.async_copy(ds2_hbm.at[row0 + c + 2], dsv.at[bs],
                                     isem[bs])
            return carry

        lax.fori_loop(0, nit // 2, pair, 0)
        plsc.subcore_barrier()
        pltpu.sync_copy(acc.at[pl.ds(sid * rpt, rpt)],
                        nbr_hbm.at[cid, pl.ds(sid * rpt, rpt)])

    return k(ext, src2d, dsts2d, zrows)


def _sc_edge_avg(nupd, src2d, dstg2d):
    n, h = nupd.shape
    nrow = src2d.shape[0]
    nit = nrow // _NW
    hw = h // 32
    rpt = _rpt(n)
    last = n - rpt * (_NS - 1)

    @functools.partial(
        pl.kernel,
        out_type=jax.ShapeDtypeStruct((nrow * _CH, h), F32),
        mesh=_mesh(),
        compiler_params=_SC_PARAMS,
        scratch_types=[
            pltpu.VMEM_SHARED((n, h), BF16),
            pltpu.VMEM((nit, _CH), jnp.int32),
            pltpu.VMEM((nit, _CH), jnp.int32),
            pltpu.VMEM((2, _CH, h), BF16),
            pltpu.VMEM((2, _CH, h), BF16),
            pltpu.VMEM((2, _CH, h), F32),
            pltpu.SemaphoreType.DMA,
            pltpu.SemaphoreType.DMA,
            pltpu.SemaphoreType.DMA,
            pltpu.SemaphoreType.DMA,
            pltpu.SemaphoreType.DMA,
            pltpu.SemaphoreType.DMA,
        ],
    )
    def k(nu_hbm, s2_hbm, d2_hbm, eo_hbm, nsp, srcv, dgv, rs, rd, eo,
          ss0, ss1, sd0, sd1, sw0, sw1):
        cid = lax.axis_index("c")
        sid = lax.axis_index("s")
        wid = sid * _NC + cid
        row0 = wid * nit
        e0 = row0 * _CH
        @pl.when(sid < _NS - 1)
        def _():
            pltpu.sync_copy(nu_hbm.at[pl.ds(sid * rpt, rpt)],
                            nsp.at[pl.ds(sid * rpt, rpt)])

        @pl.when(sid == _NS - 1)
        def _():
            pltpu.sync_copy(nu_hbm.at[pl.ds(rpt * (_NS - 1), last)],
                            nsp.at[pl.ds(rpt * (_NS - 1), last)])

        pltpu.sync_copy(s2_hbm.at[pl.ds(row0, nit)], srcv)
        pltpu.sync_copy(d2_hbm.at[pl.ds(row0, nit)], dgv)
        plsc.subcore_barrier()
        ssem = (ss0, ss1)
        dsem = (sd0, sd1)
        wsem = (sw0, sw1)
        for bs in range(2):
            pltpu.async_copy(nsp.at[srcv.at[bs]], rs.at[bs], ssem[bs])
            pltpu.async_copy(nsp.at[dgv.at[bs]], rd.at[bs], dsem[bs])

        def pair(g, carry):
            for bs in range(2):
                c = 2 * g + bs
                pltpu.make_async_copy(nsp.at[srcv.at[c]], rs.at[bs],
                                      ssem[bs]).wait()
                pltpu.make_async_copy(nsp.at[dgv.at[c]], rd.at[bs],
                                      dsem[bs]).wait()

                @pl.when(c >= 2)
                def _():
                    pltpu.make_async_copy(
                        eo.at[bs], eo_hbm.at[pl.ds(e0 + (c - 2) * _CH, _CH)],
                        wsem[bs]).wait()

                def rowf(r, c2):
                    for j in range(hw):
                        sl = pl.ds(j * 32, 32)
                        s2 = rs[bs, r, sl] + rd[bs, r, sl]
                        lo, hi = plsc.unpack(
                            s2, format=plsc.PackFormat.INTERLEAVED,
                            preferred_element_type=F32)
                        eo[bs, r, pl.ds(j * 32, 16)] = lo * 0.5
                        eo[bs, r, pl.ds(j * 32 + 16, 16)] = hi * 0.5
                    return c2

                lax.fori_loop(0, _CH, rowf, 0, unroll=2)
                pltpu.async_copy(eo.at[bs],
                                 eo_hbm.at[pl.ds(e0 + c * _CH, _CH)], wsem[bs])

                @pl.when(c + 2 < nit)
                def _():
                    pltpu.async_copy(nsp.at[srcv.at[c + 2]], rs.at[bs],
                                     ssem[bs])
                    pltpu.async_copy(nsp.at[dgv.at[c + 2]], rd.at[bs],
                                     dsem[bs])
            return carry

        lax.fori_loop(0, nit // 2, pair, 0)
        for bs in range(2):
            c = nit - 2 + bs
            pltpu.make_async_copy(eo.at[bs],
                                  eo_hbm.at[pl.ds(e0 + c * _CH, _CH)],
                                  wsem[bs]).wait()

    return k(nupd, src2d, dstg2d)


def _final_tc(p0, p1, h, n_blk=1000):
    n, hx = p0.shape

    def body(p0_ref, p1_ref, out_ref):
        s = p0_ref[:, :h] + p1_ref[:, :h]
        d = p0_ref[:, h:h + 16] + p1_ref[:, h:h + 16]
        deg = jnp.maximum(d[:, 0:1], 1.0)
        out_ref[...] = s / deg

    return pl.pallas_call(
        body,
        grid=(n // n_blk,),
        in_specs=[
            pl.BlockSpec((n_blk, hx), lambda i: (i, 0)),
            pl.BlockSpec((n_blk, hx), lambda i: (i, 0)),
        ],
        out_specs=pl.BlockSpec((n_blk, h), lambda i: (i, 0)),
        out_shape=jax.ShapeDtypeStruct((n, h), F32),
    )(p0, p1)


def kernel(node_attr, edge_idx, edge_attr, We1, be1, We2, be2,
           Wn1, bn1, Wn2, bn2):
    n, nd = node_attr.shape
    e, ed = edge_attr.shape
    h = We2.shape[1]
    src = edge_idx[0]
    dst = edge_idx[1]

    grain = _NW * _CH * 2
    e_pad = -(-e // grain) * grain
    pad = e_pad - e
    i32 = jnp.int32
    src_p = jnp.concatenate([src, jnp.zeros((pad,), i32)]).reshape(-1, _CH)
    dst_g = jnp.concatenate([dst, jnp.zeros((pad,), i32)]).reshape(-1, _CH)
    dst_s = jnp.concatenate([dst, jnp.full((pad,), n, i32)]).reshape(-1, _CH)

    a, b = _prep_tc(node_attr, We1[:nd], We1[nd:2 * nd])
    zrows = jnp.zeros((_rpt(n + 1), h), F32)

    half = e // 4
    sgrain = _NW * _CH * 2
    s_pad = -(-half // sgrain) * sgrain
    spad = s_pad - half
    aggp = None
    for lo in (0, half, 2 * half, 3 * half):
        src_s = jnp.concatenate([src[lo:lo + half],
                                 jnp.zeros((spad,), i32)]).reshape(-1, _CH)
        dstg_s = jnp.concatenate([dst[lo:lo + half],
                                  jnp.zeros((spad,), i32)]).reshape(-1, _CH)
        dsts_s = jnp.concatenate([dst[lo:lo + half],
                                  jnp.full((spad,), n, i32)]).reshape(-1, _CH)
        hpre = _sc_gather_sum(a, b, src_s, dstg_s)
        e_upd = _edge_mlp_tc(hpre, edge_attr, We1[2 * nd:],
                             be1.reshape(1, -1), We2, be2.reshape(1, -1),
                             nreal=half, ea_off=lo)
        p = _sc_scatter_add(e_upd, dsts_s, zrows, n)
        aggp = p if aggp is None else aggp + p
    n_upd, nupd_bf = _node_mlp_tc(node_attr, aggp[0, :n], aggp[1, :n],
                                  Wn1[:nd], Wn1[nd:], bn1.reshape(1, -1),
                                  Wn2, bn2.reshape(1, -1))
    hx = h + 16
    ext = jnp.concatenate(
        [n_upd, jnp.ones((n, 1), F32), jnp.zeros((n, hx - h - 1), F32)],
        axis=1)
    zrows_x = jnp.zeros((_rpt(n + 1), hx), F32)
    nbrp = _sc_nbr_scatter(ext, src_p, dst_s, zrows_x, n)
    nupd_perm = (nupd_bf.reshape(n, h // 32, 2, 16)
                 .transpose(0, 1, 3, 2).reshape(n, h))
    edge_out = _sc_edge_avg(nupd_perm, src_p, dst_g)
    node_out = _final_tc(nbrp[0, :n], nbrp[1, :n], h)
    return (node_out, edge_out[:e])


# speedup vs baseline: 1.0101x; 1.0101x over previous
import functools

import jax
import jax.numpy as jnp
from jax import lax
from jax.experimental import pallas as pl
from jax.experimental.pallas import tpu as pltpu
from jax.experimental.pallas import tpu_sc as plsc

F32 = jnp.float32
BF16 = jnp.bfloat16

_NC = 2
_NS = 16
_NW = _NC * _NS
_CH = 80

_SC_PARAMS = pltpu.CompilerParams(use_tc_tiling_on_sc=False,
                                  needs_layout_passes=False)


def _mesh():
    return plsc.VectorSubcoreMesh(core_axis_name="c", subcore_axis_name="s",
                                  num_cores=_NC, num_subcores=_NS)


def _rpt(n):
    return (-(-n // _NS) + 7) // 8 * 8


def _prep_tc(node_attr, wd, ws, n_blk=1000):
    n, nd = node_attr.shape
    h = wd.shape[1]

    def body(na_ref, wd_ref, ws_ref, a_ref, b_ref):
        na = na_ref[...]
        a_ref[...] = jnp.dot(na, wd_ref[...],
                             preferred_element_type=F32).astype(BF16)
        b_ref[...] = jnp.dot(na, ws_ref[...],
                             preferred_element_type=F32).astype(BF16)

    return pl.pallas_call(
        body,
        grid=(n // n_blk,),
        in_specs=[
            pl.BlockSpec((n_blk, nd), lambda i: (i, 0)),
            pl.BlockSpec((nd, h), lambda i: (0, 0)),
            pl.BlockSpec((nd, h), lambda i: (0, 0)),
        ],
        out_specs=[pl.BlockSpec((n_blk, h), lambda i: (i, 0))] * 2,
        out_shape=[jax.ShapeDtypeStruct((n, h), BF16)] * 2,
    )(node_attr, wd, ws)


def _sc_gather_sum(a, b, src2d, dst2d):
    n, h = a.shape
    nrow = src2d.shape[0]
    nit = nrow // _NW
    hw = h // 32
    rpt = _rpt(n)
    last = n - rpt * (_NS - 1)

    @functools.partial(
        pl.kernel,
        out_type=jax.ShapeDtypeStruct((nrow * _CH, h), BF16),
        mesh=_mesh(),
        compiler_params=_SC_PARAMS,
        scratch_types=[
            pltpu.VMEM_SHARED((n, h), BF16),
            pltpu.VMEM_SHARED((n, h), BF16),
            pltpu.VMEM((nit, _CH), jnp.int32),
            pltpu.VMEM((2, _CH), jnp.int32),
            pltpu.VMEM((2, _CH, h), BF16),
            pltpu.VMEM((2, _CH, h), BF16),
            pltpu.VMEM((2, _CH, h), BF16),
            pltpu.SemaphoreType.DMA,
            pltpu.SemaphoreType.DMA,
            pltpu.SemaphoreType.DMA,
            pltpu.SemaphoreType.DMA,
            pltpu.SemaphoreType.DMA,
            pltpu.SemaphoreType.DMA,
            pltpu.SemaphoreType.DMA,
            pltpu.SemaphoreType.DMA,
        ],
    )
    def k(a_hbm, b_hbm, s2_hbm, d2_hbm, out_hbm, asp, bsp, srcv, dsv,
          ra, rb, ro, sa0, sa1, sb0, sb1, sw0, sw1, si0, si1):
        cid = lax.axis_index("c")
        sid = lax.axis_index("s")
        wid = sid * _NC + cid
        row0 = wid * nit
        e0 = row0 * _CH

        @pl.when(sid < _NS - 1)
        def _():
            pltpu.sync_copy(a_hbm.at[pl.ds(sid * rpt, rpt)],
                            asp.at[pl.ds(sid * rpt, rpt)])
            pltpu.sync_copy(b_hbm.at[pl.ds(sid * rpt, rpt)],
                            bsp.at[pl.ds(sid * rpt, rpt)])

        @pl.when(sid == _NS - 1)
        def _():
            pltpu.sync_copy(a_hbm.at[pl.ds(rpt * (_NS - 1), last)],
                            asp.at[pl.ds(rpt * (_NS - 1), last)])
            pltpu.sync_copy(b_hbm.at[pl.ds(rpt * (_NS - 1), last)],
                            bsp.at[pl.ds(rpt * (_NS - 1), last)])

        pltpu.sync_copy(s2_hbm.at[pl.ds(row0, nit)], srcv)
        for bs in range(2):
            pltpu.sync_copy(d2_hbm.at[row0 + bs], dsv.at[bs])
        plsc.subcore_barrier()
        sa = (sa0, sa1)
        sb = (sb0, sb1)
        sw = (sw0, sw1)
        si = (si0, si1)
        for bs in range(2):
            pltpu.async_copy(asp.at[dsv.at[bs]], ra.at[bs], sa[bs])
            pltpu.async_copy(bsp.at[srcv.at[bs]], rb.at[bs], sb[bs])

        def pair(g, carry):
            for bs in range(2):
                c = 2 * g + bs
                pltpu.make_async_copy(asp.at[dsv.at[bs]], ra.at[bs],
                                      sa[bs]).wait()
                pltpu.make_async_copy(bsp.at[srcv.at[c]], rb.at[bs],
                                      sb[bs]).wait()

                @pl.when(c + 2 < nit)
                def _():
                    pltpu.async_copy(d2_hbm.at[row0 + c + 2], dsv.at[bs],
                                     si[bs])

                @pl.when(c >= 2)
                def _():
                    pltpu.make_async_copy(
                        ro.at[bs], out_hbm.at[pl.ds(e0 + (c - 2) * _CH, _CH)],
                        sw[bs]).wait()

                def rowf(r, c2):
                    for j in range(hw):
                        sl = pl.ds(j * 32, 32)
                        ro[bs, r, sl] = ra[bs, r, sl] + rb[bs, r, sl]
                    return c2

                lax.fori_loop(0, _CH, rowf, 0, unroll=2)
                pltpu.async_copy(ro.at[bs],
                                 out_hbm.at[pl.ds(e0 + c * _CH, _CH)], sw[bs])

                @pl.when(c + 2 < nit)
                def _():
                    pltpu.make_async_copy(d2_hbm.at[row0 + c + 2],
                                          dsv.at[bs], si[bs]).wait()
                    pltpu.async_copy(asp.at[dsv.at[bs]], ra.at[bs], sa[bs])
                    pltpu.async_copy(bsp.at[srcv.at[c + 2]], rb.at[bs],
                                     sb[bs])
            return carry

        lax.fori_loop(0, nit // 2, pair, 0)
        for bs in range(2):
            c = nit - 2 + bs
            pltpu.make_async_copy(ro.at[bs],
                                  out_hbm.at[pl.ds(e0 + c * _CH, _CH)],
                                  sw[bs]).wait()

    return k(a, b, src2d, dst2d)


def _edge_mlp_tc(hpre, ea, w1e, b1, w2, b2, nreal, ea_off, e_blk=2000):
    e, h = hpre.shape
    ed = ea.shape[1]
    off = ea_off // e_blk

    def body(hp_ref, ea_ref, w1_ref, b1_ref, w2_ref, b2_ref, out_ref):
        hh = hp_ref[...].astype(F32) + jnp.dot(
            ea_ref[...], w1_ref[...],
            preferred_element_type=F32) + b1_ref[...]
        hh = jnp.maximum(hh, 0.0)
        out_ref[...] = jnp.dot(hh, w2_ref[...],
                               preferred_element_type=F32) + b2_ref[...]

    return pl.pallas_call(
        body,
        grid=(nreal // e_blk,),
        in_specs=[
            pl.BlockSpec((e_blk, h), lambda i: (i, 0)),
            pl.BlockSpec((e_blk, ed), lambda i: (i + off, 0)),
            pl.BlockSpec((ed, h), lambda i: (0, 0)),
            pl.BlockSpec((1, h), lambda i: (0, 0)),
            pl.BlockSpec((h, h), lambda i: (0, 0)),
            pl.BlockSpec((1, h), lambda i: (0, 0)),
        ],
        out_specs=pl.BlockSpec((e_blk, h), lambda i: (i, 0)),
        out_shape=jax.ShapeDtypeStruct((e, h), F32),
    )(hpre, ea, w1e, b1, w2, b2)


def _sc_scatter_add(e_upd, dst2d, zrows, n):
    e, h = e_upd.shape
    nrow = dst2d.shape[0]
    nit = nrow // _NW
    rpt = _rpt(n + 1)
    npad = rpt * _NS

    @functools.partial(
        pl.kernel,
        out_type=jax.ShapeDtypeStruct((_NC, npad, h), F32),
        mesh=_mesh(),
        compiler_params=_SC_PARAMS,
        scratch_types=[
            pltpu.VMEM_SHARED((npad, h), F32),
            pltpu.VMEM((nit, _CH), jnp.int32),
            pltpu.VMEM((2, _CH, h), F32),
            pltpu.SemaphoreType.DMA,
            pltpu.SemaphoreType.DMA,
            pltpu.SemaphoreType.DMA,
            pltpu.SemaphoreType.DMA,
        ],
    )
    def k(eu_hbm, d2_hbm, z_hbm, out_hbm, acc, dstv, buf, sr0, sr1, ss0, ss1):
        cid = lax.axis_index("c")
        sid = lax.axis_index("s")
        wid = sid * _NC + cid
        row0 = wid * nit
        e0 = row0 * _CH
        pltpu.sync_copy(d2_hbm.at[pl.ds(row0, nit)], dstv)
        pltpu.sync_copy(z_hbm, acc.at[pl.ds(sid * rpt, rpt)])
        plsc.subcore_barrier()
        sr = (sr0, sr1)
        ss = (ss0, ss1)
        for bs in range(2):
            pltpu.async_copy(eu_hbm.at[pl.ds(e0 + bs * _CH, _CH)], buf.at[bs],
                             sr[bs])

        def pair(g, carry):
            for bs in range(2):
                c = 2 * g + bs
                pltpu.make_async_copy(eu_hbm.at[pl.ds(e0 + c * _CH, _CH)],
                                      buf.at[bs], sr[bs]).wait()
                pltpu.async_copy(buf.at[bs], acc.at[dstv.at[c]], ss[bs],
                                 add=True)
                pltpu.make_async_copy(buf.at[bs], acc.at[dstv.at[c]],
                                      ss[bs]).wait()

                @pl.when(c + 2 < nit)
                def _():
                    pltpu.async_copy(eu_hbm.at[pl.ds(e0 + (c + 2) * _CH, _CH)],
                                     buf.at[bs], sr[bs])
            return carry

        lax.fori_loop(0, nit // 2, pair, 0)
        plsc.subcore_barrier()
        pltpu.sync_copy(acc.at[pl.ds(sid * rpt, rpt)],
                        out_hbm.at[cid, pl.ds(sid * rpt, rpt)])

    return k(e_upd, dst2d, zrows)


def _node_mlp_tc(na, a0, a1, w1a, w1b, b1, w2, b2, n_blk=1000):
    n, nd = na.shape
    h = w1a.shape[1]

    def body(na_ref, a0_ref, a1_ref, w1a_ref, w1b_ref, b1_ref, w2_ref, b2_ref,
             out_ref, obf_ref):
        agg = a0_ref[...] + a1_ref[...]
        hh = jnp.dot(na_ref[...], w1a_ref[...], preferred_element_type=F32)
        hh = hh + jnp.dot(agg, w1b_ref[...], preferred_element_type=F32)
        hh = jnp.maximum(hh + b1_ref[...], 0.0)
        nu = jnp.dot(hh, w2_ref[...], preferred_element_type=F32) + b2_ref[...]
        out_ref[...] = nu
        obf_ref[...] = nu.astype(BF16)

    return pl.pallas_call(
        body,
        grid=(n // n_blk,),
        in_specs=[
            pl.BlockSpec((n_blk, nd), lambda i: (i, 0)),
            pl.BlockSpec((n_blk, h), lambda i: (i, 0)),
            pl.BlockSpec((n_blk, h), lambda i: (i, 0)),
            pl.BlockSpec((nd, h), lambda i: (0, 0)),
            pl.BlockSpec((h, h), lambda i: (0, 0)),
            pl.BlockSpec((1, h), lambda i: (0, 0)),
            pl.BlockSpec((h, h), lambda i: (0, 0)),
            pl.BlockSpec((1, h), lambda i: (0, 0)),
        ],
        out_specs=[pl.BlockSpec((n_blk, h), lambda i: (i, 0))] * 2,
        out_shape=[jax.ShapeDtypeStruct((n, h), F32),
                   jax.ShapeDtypeStruct((n, h), BF16)],
    )(na, a0, a1, w1a, w1b, b1, w2, b2)


def _sc_nbr_scatter(ext, src2d, dsts2d, zrows, n):
    hx = ext.shape[1]
    nrow = src2d.shape[0]
    nit = nrow // _NW
    rpt = _rpt(n + 1)
    npad = rpt * _NS

    @functools.partial(
        pl.kernel,
        out_type=jax.ShapeDtypeStruct((_NC, npad, hx), F32),
        mesh=_mesh(),
        compiler_params=_SC_PARAMS,
        scratch_types=[
            pltpu.VMEM_SHARED((npad, hx), F32),
            pltpu.VMEM((nit, _CH), jnp.int32),
            pltpu.VMEM((2, _CH), jnp.int32),
            pltpu.VMEM((2, _CH, hx), F32),
            pltpu.SemaphoreType.DMA,
            pltpu.SemaphoreType.DMA,
            pltpu.SemaphoreType.DMA,
            pltpu.SemaphoreType.DMA,
            pltpu.SemaphoreType.DMA,
            pltpu.SemaphoreType.DMA,
        ],
    )
    def k(ext_hbm, s2_hbm, ds2_hbm, z_hbm, nbr_hbm,
          acc, srcv, dsv, rs, sg0, sg1, si0, si1, sc0, sc1):
        cid = lax.axis_index("c")
        sid = lax.axis_index("s")
        wid = sid * _NC + cid
        row0 = wid * nit
        pltpu.sync_copy(s2_hbm.at[pl.ds(row0, nit)], srcv)
        pltpu.sync_copy(z_hbm, acc.at[pl.ds(sid * rpt, rpt)])
        plsc.subcore_barrier()
        gsem = (sg0, sg1)
        isem = (si0, si1)
        csem = (sc0, sc1)
        for bs in range(2):
            pltpu.async_copy(ext_hbm.at[srcv.at[bs]], rs.at[bs], gsem[bs])
            pltpu.async_copy(ds2_hbm.at[row0 + bs], dsv.at[bs], isem[bs])

        def pair(g, carry):
            for bs in range(2):
                c = 2 * g + bs
                pltpu.make_async_copy(ext_hbm.at[srcv.at[c]], rs.at[bs],
                                      gsem[bs]).wait()
                pltpu.make_async_copy(ds2_hbm.at[row0 + c], dsv.at[bs],
                                      isem[bs]).wait()
                pltpu.async_copy(rs.at[bs], acc.at[dsv.at[bs]], csem[bs],
                                 add=True)
                pltpu.make_async_copy(rs.at[bs], acc.at[dsv.at[bs]],
                                      csem[bs]).wait()

                @pl.when(c + 2 < nit)
                def _():
                    pltpu.async_copy(ext_hbm.at[srcv.at[c + 2]], rs.at[bs],
                                     gsem[bs])
                    pltpu.async_copy(ds2_hbm.at[row0 + c + 2], dsv.at[bs],
                                     isem[bs])
            return carry

        lax.fori_loop(0, nit // 2, pair, 0)
        plsc.subcore_barrier()
        pltpu.sync_copy(acc.at[pl.ds(sid * rpt, rpt)],
                        nbr_hbm.at[cid, pl.ds(sid * rpt, rpt)])

    return k(ext, src2d, dsts2d, zrows)


def _sc_edge_avg(nupd, src2d, dstg2d):
    n, h = nupd.shape
    nrow = src2d.shape[0]
    nit = nrow // _NW
    hw = h // 32
    rpt = _rpt(n)
    last = n - rpt * (_NS - 1)

    @functools.partial(
        pl.kernel,
        out_type=jax.ShapeDtypeStruct((nrow * _CH, h), F32),
        mesh=_mesh(),
        compiler_params=_SC_PARAMS,
        scratch_types=[
            pltpu.VMEM_SHARED((n, h), BF16),
            pltpu.VMEM((nit, _CH), jnp.int32),
            pltpu.VMEM((nit, _CH), jnp.int32),
            pltpu.VMEM((2, _CH, h), BF16),
            pltpu.VMEM((2, _CH, h), BF16),
            pltpu.VMEM((2, _CH, h), F32),
            pltpu.SemaphoreType.DMA,
            pltpu.SemaphoreType.DMA,
            pltpu.SemaphoreType.DMA,
            pltpu.SemaphoreType.DMA,
            pltpu.SemaphoreType.DMA,
            pltpu.SemaphoreType.DMA,
        ],
    )
    def k(nu_hbm, s2_hbm, d2_hbm, eo_hbm, nsp, srcv, dgv, rs, rd, eo,
          ss0, ss1, sd0, sd1, sw0, sw1):
        cid = lax.axis_index("c")
        sid = lax.axis_index("s")
        wid = sid * _NC + cid
        row0 = wid * nit
        e0 = row0 * _CH
        @pl.when(sid < _NS - 1)
        def _():
            pltpu.sync_copy(nu_hbm.at[pl.ds(sid * rpt, rpt)],
                            nsp.at[pl.ds(sid * rpt, rpt)])

        @pl.when(sid == _NS - 1)
        def _():
            pltpu.sync_copy(nu_hbm.at[pl.ds(rpt * (_NS - 1), last)],
                            nsp.at[pl.ds(rpt * (_NS - 1), last)])

        pltpu.sync_copy(s2_hbm.at[pl.ds(row0, nit)], srcv)
        pltpu.sync_copy(d2_hbm.at[pl.ds(row0, nit)], dgv)
        plsc.subcore_barrier()
        ssem = (ss0, ss1)
        dsem = (sd0, sd1)
        wsem = (sw0, sw1)
        for bs in range(2):
            pltpu.async_copy(nsp.at[srcv.at[bs]], rs.at[bs], ssem[bs])
            pltpu.async_copy(nsp.at[dgv.at[bs]], rd.at[bs], dsem[bs])

        def pair(g, carry):
            for bs in range(2):
                c = 2 * g + bs
                pltpu.make_async_copy(nsp.at[srcv.at[c]], rs.at[bs],
                                      ssem[bs]).wait()
                pltpu.make_async_copy(nsp.at[dgv.at[c]], rd.at[bs],
                                      dsem[bs]).wait()

                @pl.when(c >= 2)
                def _():
                    pltpu.make_async_copy(
                        eo.at[bs], eo_hbm.at[pl.ds(e0 + (c - 2) * _CH, _CH)],
                        wsem[bs]).wait()

                def rowf(r, c2):
                    for j in range(hw):
                        sl = pl.ds(j * 32, 32)
                        s2 = rs[bs, r, sl] + rd[bs, r, sl]
                        lo, hi = plsc.unpack(
                            s2, format=plsc.PackFormat.INTERLEAVED,
                            preferred_element_type=F32)
                        eo[bs, r, pl.ds(j * 32, 16)] = lo * 0.5
                        eo[bs, r, pl.ds(j * 32 + 16, 16)] = hi * 0.5
                    return c2

                lax.fori_loop(0, _CH, rowf, 0, unroll=2)
                pltpu.async_copy(eo.at[bs],
                                 eo_hbm.at[pl.ds(e0 + c * _CH, _CH)], wsem[bs])

                @pl.when(c + 2 < nit)
                def _():
                    pltpu.async_copy(nsp.at[srcv.at[c + 2]], rs.at[bs],
                                     ssem[bs])
                    pltpu.async_copy(nsp.at[dgv.at[c + 2]], rd.at[bs],
                                     dsem[bs])
            return carry

        lax.fori_loop(0, nit // 2, pair, 0)
        for bs in range(2):
            c = nit - 2 + bs
            pltpu.make_async_copy(eo.at[bs],
                                  eo_hbm.at[pl.ds(e0 + c * _CH, _CH)],
                                  wsem[bs]).wait()

    return k(nupd, src2d, dstg2d)


def _final_tc(p0, p1, h, n_blk=1000):
    n, hx = p0.shape

    def body(p0_ref, p1_ref, out_ref):
        s = p0_ref[:, :h] + p1_ref[:, :h]
        d = p0_ref[:, h:h + 16] + p1_ref[:, h:h + 16]
        deg = jnp.maximum(d[:, 0:1], 1.0)
        out_ref[...] = s / deg

    return pl.pallas_call(
        body,
        grid=(n // n_blk,),
        in_specs=[
            pl.BlockSpec((n_blk, hx), lambda i: (i, 0)),
            pl.BlockSpec((n_blk, hx), lambda i: (i, 0)),
        ],
        out_specs=pl.BlockSpec((n_blk, h), lambda i: (i, 0)),
        out_shape=jax.ShapeDtypeStruct((n, h), F32),
    )(p0, p1)


def kernel(node_attr, edge_idx, edge_attr, We1, be1, We2, be2,
           Wn1, bn1, Wn2, bn2):
    n, nd = node_attr.shape
    e, ed = edge_attr.shape
    h = We2.shape[1]
    src = edge_idx[0]
    dst = edge_idx[1]

    grain = _NW * _CH * 2
    e_pad = -(-e // grain) * grain
    pad = e_pad - e
    i32 = jnp.int32
    src_p = jnp.concatenate([src, jnp.zeros((pad,), i32)]).reshape(-1, _CH)
    dst_g = jnp.concatenate([dst, jnp.zeros((pad,), i32)]).reshape(-1, _CH)
    dst_s = jnp.concatenate([dst, jnp.full((pad,), n, i32)]).reshape(-1, _CH)

    a, b = _prep_tc(node_attr, We1[:nd], We1[nd:2 * nd])
    zrows = jnp.zeros((_rpt(n + 1), h), F32)

    half = e // 2
    sgrain = _NW * _CH * 2
    s_pad = -(-half // sgrain) * sgrain
    spad = s_pad - half
    aggp = None
    for lo in (0, half):
        src_s = jnp.concatenate([src[lo:lo + half],
                                 jnp.zeros((spad,), i32)]).reshape(-1, _CH)
        dstg_s = jnp.concatenate([dst[lo:lo + half],
                                  jnp.zeros((spad,), i32)]).reshape(-1, _CH)
        dsts_s = jnp.concatenate([dst[lo:lo + half],
                                  jnp.full((spad,), n, i32)]).reshape(-1, _CH)
        hpre = _sc_gather_sum(a, b, src_s, dstg_s)
        e_upd = _edge_mlp_tc(hpre, edge_attr, We1[2 * nd:],
                             be1.reshape(1, -1), We2, be2.reshape(1, -1),
                             nreal=half, ea_off=lo)
        p = _sc_scatter_add(e_upd, dsts_s, zrows, n)
        aggp = p if aggp is None else aggp + p
    n_upd, nupd_bf = _node_mlp_tc(node_attr, aggp[0, :n], aggp[1, :n],
                                  Wn1[:nd], Wn1[nd:], bn1.reshape(1, -1),
                                  Wn2, bn2.reshape(1, -1))
    hx = h + 16
    ext = jnp.concatenate(
        [n_upd, jnp.ones((n, 1), F32), jnp.zeros((n, hx - h - 1), F32)],
        axis=1)
    zrows_x = jnp.zeros((_rpt(n + 1), hx), F32)
    nbrp = _sc_nbr_scatter(ext, src_p, dst_s, zrows_x, n)
    nupd_perm = (nupd_bf.reshape(n, h // 32, 2, 16)
                 .transpose(0, 1, 3, 2).reshape(n, h))
    edge_out = _sc_edge_avg(nupd_perm, src_p, dst_g)
    node_out = _final_tc(nbrp[0, :n], nbrp[1, :n], h)
    return (node_out, edge_out[:e])


# speedup vs baseline: 1.0124x; 1.0023x over previous
import functools

import jax
import jax.numpy as jnp
from jax import lax
from jax.experimental import pallas as pl
from jax.experimental.pallas import tpu as pltpu
from jax.experimental.pallas import tpu_sc as plsc

F32 = jnp.float32
BF16 = jnp.bfloat16

_NC = 2
_NS = 16
_NW = _NC * _NS
_CH = 80

_SC_PARAMS = pltpu.CompilerParams(use_tc_tiling_on_sc=False,
                                  needs_layout_passes=False)


def _mesh():
    return plsc.VectorSubcoreMesh(core_axis_name="c", subcore_axis_name="s",
                                  num_cores=_NC, num_subcores=_NS)


def _rpt(n):
    return (-(-n // _NS) + 7) // 8 * 8


def _prep_tc(node_attr, wd, ws, n_blk=1000):
    n, nd = node_attr.shape
    h = wd.shape[1]

    def body(na_ref, wd_ref, ws_ref, a_ref, b_ref):
        na = na_ref[...]
        a_ref[...] = jnp.dot(na, wd_ref[...],
                             preferred_element_type=F32).astype(BF16)
        b_ref[...] = jnp.dot(na, ws_ref[...],
                             preferred_element_type=F32).astype(BF16)

    return pl.pallas_call(
        body,
        grid=(n // n_blk,),
        in_specs=[
            pl.BlockSpec((n_blk, nd), lambda i: (i, 0)),
            pl.BlockSpec((nd, h), lambda i: (0, 0)),
            pl.BlockSpec((nd, h), lambda i: (0, 0)),
        ],
        out_specs=[pl.BlockSpec((n_blk, h), lambda i: (i, 0))] * 2,
        out_shape=[jax.ShapeDtypeStruct((n, h), BF16)] * 2,
    )(node_attr, wd, ws)


def _sc_gather_sum(a, b, src2d, dst2d):
    n, h = a.shape
    nrow = src2d.shape[0]
    nit = nrow // _NW
    hw = h // 32
    rpt = _rpt(n)
    last = n - rpt * (_NS - 1)

    @functools.partial(
        pl.kernel,
        out_type=jax.ShapeDtypeStruct((nrow * _CH, h), BF16),
        mesh=_mesh(),
        compiler_params=_SC_PARAMS,
        scratch_types=[
            pltpu.VMEM_SHARED((n, h), BF16),
            pltpu.VMEM_SHARED((n, h), BF16),
            pltpu.VMEM((nit, _CH), jnp.int32),
            pltpu.VMEM((2, _CH), jnp.int32),
            pltpu.VMEM((2, _CH, h), BF16),
            pltpu.VMEM((2, _CH, h), BF16),
            pltpu.VMEM((2, _CH, h), BF16),
            pltpu.SemaphoreType.DMA,
            pltpu.SemaphoreType.DMA,
            pltpu.SemaphoreType.DMA,
            pltpu.SemaphoreType.DMA,
            pltpu.SemaphoreType.DMA,
            pltpu.SemaphoreType.DMA,
            pltpu.SemaphoreType.DMA,
            pltpu.SemaphoreType.DMA,
        ],
    )
    def k(a_hbm, b_hbm, s2_hbm, d2_hbm, out_hbm, asp, bsp, srcv, dsv,
          ra, rb, ro, sa0, sa1, sb0, sb1, sw0, sw1, si0, si1):
        cid = lax.axis_index("c")
        sid = lax.axis_index("s")
        wid = sid * _NC + cid
        row0 = wid * nit
        e0 = row0 * _CH

        @pl.when(sid < _NS - 1)
        def _():
            pltpu.sync_copy(a_hbm.at[pl.ds(sid * rpt, rpt)],
                            asp.at[pl.ds(sid * rpt, rpt)])
            pltpu.sync_copy(b_hbm.at[pl.ds(sid * rpt, rpt)],
                            bsp.at[pl.ds(sid * rpt, rpt)])

        @pl.when(sid == _NS - 1)
        def _():
            pltpu.sync_copy(a_hbm.at[pl.ds(rpt * (_NS - 1), last)],
                            asp.at[pl.ds(rpt * (_NS - 1), last)])
            pltpu.sync_copy(b_hbm.at[pl.ds(rpt * (_NS - 1), last)],
                            bsp.at[pl.ds(rpt * (_NS - 1), last)])

        pltpu.sync_copy(s2_hbm.at[pl.ds(row0, nit)], srcv)
        for bs in range(2):
            pltpu.sync_copy(d2_hbm.at[row0 + bs], dsv.at[bs])
        plsc.subcore_barrier()
        sa = (sa0, sa1)
        sb = (sb0, sb1)
        sw = (sw0, sw1)
        si = (si0, si1)
        for bs in range(2):
            pltpu.async_copy(asp.at[dsv.at[bs]], ra.at[bs], sa[bs])
            pltpu.async_copy(bsp.at[srcv.at[bs]], rb.at[bs], sb[bs])

        def pair(g, carry):
            for bs in range(2):
                c = 2 * g + bs
                pltpu.make_async_copy(asp.at[dsv.at[bs]], ra.at[bs],
                                      sa[bs]).wait()
                pltpu.make_async_copy(bsp.at[srcv.at[c]], rb.at[bs],
                                      sb[bs]).wait()

                @pl.when(c + 2 < nit)
                def _():
                    pltpu.async_copy(d2_hbm.at[row0 + c + 2], dsv.at[bs],
                                     si[bs])

                @pl.when(c >= 2)
                def _():
                    pltpu.make_async_copy(
                        ro.at[bs], out_hbm.at[pl.ds(e0 + (c - 2) * _CH, _CH)],
                        sw[bs]).wait()

                def rowf(r, c2):
                    for j in range(hw):
                        sl = pl.ds(j * 32, 32)
                        ro[bs, r, sl] = ra[bs, r, sl] + rb[bs, r, sl]
                    return c2

                lax.fori_loop(0, _CH, rowf, 0, unroll=4)
                pltpu.async_copy(ro.at[bs],
                                 out_hbm.at[pl.ds(e0 + c * _CH, _CH)], sw[bs])

                @pl.when(c + 2 < nit)
                def _():
                    pltpu.make_async_copy(d2_hbm.at[row0 + c + 2],
                                          dsv.at[bs], si[bs]).wait()
                    pltpu.async_copy(asp.at[dsv.at[bs]], ra.at[bs], sa[bs])
                    pltpu.async_copy(bsp.at[srcv.at[c + 2]], rb.at[bs],
                                     sb[bs])
            return carry

        lax.fori_loop(0, nit // 2, pair, 0)
        for bs in range(2):
            c = nit - 2 + bs
            pltpu.make_async_copy(ro.at[bs],
                                  out_hbm.at[pl.ds(e0 + c * _CH, _CH)],
                                  sw[bs]).wait()

    return k(a, b, src2d, dst2d)


def _edge_mlp_tc(hpre, ea, w1e, b1, w2, b2, nreal, ea_off, e_blk=2000):
    e, h = hpre.shape
    ed = ea.shape[1]
    off = ea_off // e_blk

    def body(hp_ref, ea_ref, w1_ref, b1_ref, w2_ref, b2_ref, out_ref):
        hh = hp_ref[...].astype(F32) + jnp.dot(
            ea_ref[...], w1_ref[...],
            preferred_element_type=F32) + b1_ref[...]
        hh = jnp.maximum(hh, 0.0)
        out_ref[...] = jnp.dot(hh, w2_ref[...],
                               preferred_element_type=F32) + b2_ref[...]

    return pl.pallas_call(
        body,
        grid=(nreal // e_blk,),
        in_specs=[
            pl.BlockSpec((e_blk, h), lambda i: (i, 0)),
            pl.BlockSpec((e_blk, ed), lambda i: (i + off, 0)),
            pl.BlockSpec((ed, h), lambda i: (0, 0)),
            pl.BlockSpec((1, h), lambda i: (0, 0)),
            pl.BlockSpec((h, h), lambda i: (0, 0)),
            pl.BlockSpec((1, h), lambda i: (0, 0)),
        ],
        out_specs=pl.BlockSpec((e_blk, h), lambda i: (i, 0)),
        out_shape=jax.ShapeDtypeStruct((e, h), F32),
    )(hpre, ea, w1e, b1, w2, b2)


def _sc_scatter_add(e_upd, dst2d, zrows, n):
    e, h = e_upd.shape
    nrow = dst2d.shape[0]
    nit = nrow // _NW
    rpt = _rpt(n + 1)
    npad = rpt * _NS

    @functools.partial(
        pl.kernel,
        out_type=jax.ShapeDtypeStruct((_NC, npad, h), F32),
        mesh=_mesh(),
        compiler_params=_SC_PARAMS,
        scratch_types=[
            pltpu.VMEM_SHARED((npad, h), F32),
            pltpu.VMEM((nit, _CH), jnp.int32),
            pltpu.VMEM((2, _CH, h), F32),
            pltpu.SemaphoreType.DMA,
            pltpu.SemaphoreType.DMA,
            pltpu.SemaphoreType.DMA,
            pltpu.SemaphoreType.DMA,
        ],
    )
    def k(eu_hbm, d2_hbm, z_hbm, out_hbm, acc, dstv, buf, sr0, sr1, ss0, ss1):
        cid = lax.axis_index("c")
        sid = lax.axis_index("s")
        wid = sid * _NC + cid
        row0 = wid * nit
        e0 = row0 * _CH
        pltpu.sync_copy(d2_hbm.at[pl.ds(row0, nit)], dstv)
        pltpu.sync_copy(z_hbm, acc.at[pl.ds(sid * rpt, rpt)])
        plsc.subcore_barrier()
        sr = (sr0, sr1)
        ss = (ss0, ss1)
        for bs in range(2):
            pltpu.async_copy(eu_hbm.at[pl.ds(e0 + bs * _CH, _CH)], buf.at[bs],
                             sr[bs])

        def pair(g, carry):
            for bs in range(2):
                c = 2 * g + bs
                pltpu.make_async_copy(eu_hbm.at[pl.ds(e0 + c * _CH, _CH)],
                                      buf.at[bs], sr[bs]).wait()
                pltpu.async_copy(buf.at[bs], acc.at[dstv.at[c]], ss[bs],
                                 add=True)
                pltpu.make_async_copy(buf.at[bs], acc.at[dstv.at[c]],
                                      ss[bs]).wait()

                @pl.when(c + 2 < nit)
                def _():
                    pltpu.async_copy(eu_hbm.at[pl.ds(e0 + (c + 2) * _CH, _CH)],
                                     buf.at[bs], sr[bs])
            return carry

        lax.fori_loop(0, nit // 2, pair, 0)
        plsc.subcore_barrier()
        pltpu.sync_copy(acc.at[pl.ds(sid * rpt, rpt)],
                        out_hbm.at[cid, pl.ds(sid * rpt, rpt)])

    return k(e_upd, dst2d, zrows)


def _node_mlp_tc(na, a0, a1, w1a, w1b, b1, w2, b2, n_blk=1000):
    n, nd = na.shape
    h = w1a.shape[1]

    def body(na_ref, a0_ref, a1_ref, w1a_ref, w1b_ref, b1_ref, w2_ref, b2_ref,
             out_ref, obf_ref):
        agg = a0_ref[...] + a1_ref[...]
        hh = jnp.dot(na_ref[...], w1a_ref[...], preferred_element_type=F32)
        hh = hh + jnp.dot(agg, w1b_ref[...], preferred_element_type=F32)
        hh = jnp.maximum(hh + b1_ref[...], 0.0)
        nu = jnp.dot(hh, w2_ref[...], preferred_element_type=F32) + b2_ref[...]
        out_ref[...] = nu
        obf_ref[...] = nu.astype(BF16)

    return pl.pallas_call(
        body,
        grid=(n // n_blk,),
        in_specs=[
            pl.BlockSpec((n_blk, nd), lambda i: (i, 0)),
            pl.BlockSpec((n_blk, h), lambda i: (i, 0)),
            pl.BlockSpec((n_blk, h), lambda i: (i, 0)),
            pl.BlockSpec((nd, h), lambda i: (0, 0)),
            pl.BlockSpec((h, h), lambda i: (0, 0)),
            pl.BlockSpec((1, h), lambda i: (0, 0)),
            pl.BlockSpec((h, h), lambda i: (0, 0)),
            pl.BlockSpec((1, h), lambda i: (0, 0)),
        ],
        out_specs=[pl.BlockSpec((n_blk, h), lambda i: (i, 0))] * 2,
        out_shape=[jax.ShapeDtypeStruct((n, h), F32),
                   jax.ShapeDtypeStruct((n, h), BF16)],
    )(na, a0, a1, w1a, w1b, b1, w2, b2)


def _sc_nbr_scatter(ext, src2d, dsts2d, zrows, n):
    hx = ext.shape[1]
    nrow = src2d.shape[0]
    nit = nrow // _NW
    rpt = _rpt(n + 1)
    npad = rpt * _NS

    @functools.partial(
        pl.kernel,
        out_type=jax.ShapeDtypeStruct((_NC, npad, hx), F32),
        mesh=_mesh(),
        compiler_params=_SC_PARAMS,
        scratch_types=[
            pltpu.VMEM_SHARED((npad, hx), F32),
            pltpu.VMEM((nit, _CH), jnp.int32),
            pltpu.VMEM((2, _CH), jnp.int32),
            pltpu.VMEM((2, _CH, hx), F32),
            pltpu.SemaphoreType.DMA,
            pltpu.SemaphoreType.DMA,
            pltpu.SemaphoreType.DMA,
            pltpu.SemaphoreType.DMA,
            pltpu.SemaphoreType.DMA,
            pltpu.SemaphoreType.DMA,
        ],
    )
    def k(ext_hbm, s2_hbm, ds2_hbm, z_hbm, nbr_hbm,
          acc, srcv, dsv, rs, sg0, sg1, si0, si1, sc0, sc1):
        cid = lax.axis_index("c")
        sid = lax.axis_index("s")
        wid = sid * _NC + cid
        row0 = wid * nit
        pltpu.sync_copy(s2_hbm.at[pl.ds(row0, nit)], srcv)
        pltpu.sync_copy(z_hbm, acc.at[pl.ds(sid * rpt, rpt)])
        plsc.subcore_barrier()
        gsem = (sg0, sg1)
        isem = (si0, si1)
        csem = (sc0, sc1)
        for bs in range(2):
            pltpu.async_copy(ext_hbm.at[srcv.at[bs]], rs.at[bs], gsem[bs])
            pltpu.async_copy(ds2_hbm.at[row0 + bs], dsv.at[bs], isem[bs])

        def pair(g, carry):
            for bs in range(2):
                c = 2 * g + bs
                pltpu.make_async_copy(ext_hbm.at[srcv.at[c]], rs.at[bs],
                                      gsem[bs]).wait()
                pltpu.make_async_copy(ds2_hbm.at[row0 + c], dsv.at[bs],
                                      isem[bs]).wait()
                pltpu.async_copy(rs.at[bs], acc.at[dsv.at[bs]], csem[bs],
                                 add=True)
                pltpu.make_async_copy(rs.at[bs], acc.at[dsv.at[bs]],
                                      csem[bs]).wait()

                @pl.when(c + 2 < nit)
                def _():
                    pltpu.async_copy(ext_hbm.at[srcv.at[c + 2]], rs.at[bs],
                                     gsem[bs])
                    pltpu.async_copy(ds2_hbm.at[row0 + c + 2], dsv.at[bs],
                                     isem[bs])
            return carry

        lax.fori_loop(0, nit // 2, pair, 0)
        plsc.subcore_barrier()
        pltpu.sync_copy(acc.at[pl.ds(sid * rpt, rpt)],
                        nbr_hbm.at[cid, pl.ds(sid * rpt, rpt)])

    return k(ext, src2d, dsts2d, zrows)


def _sc_edge_avg(nupd, src2d, dstg2d):
    n, h = nupd.shape
    nrow = src2d.shape[0]
    nit = nrow // _NW
    hw = h // 32
    rpt = _rpt(n)
    last = n - rpt * (_NS - 1)

    @functools.partial(
        pl.kernel,
        out_type=jax.ShapeDtypeStruct((nrow * _CH, h), F32),
        mesh=_mesh(),
        compiler_params=_SC_PARAMS,
        scratch_types=[
            pltpu.VMEM_SHARED((n, h), BF16),
            pltpu.VMEM((nit, _CH), jnp.int32),
            pltpu.VMEM((nit, _CH), jnp.int32),
            pltpu.VMEM((2, _CH, h), BF16),
            pltpu.VMEM((2, _CH, h), BF16),
            pltpu.VMEM((2, _CH, h), F32),
            pltpu.SemaphoreType.DMA,
            pltpu.SemaphoreType.DMA,
            pltpu.SemaphoreType.DMA,
            pltpu.SemaphoreType.DMA,
            pltpu.SemaphoreType.DMA,
            pltpu.SemaphoreType.DMA,
        ],
    )
    def k(nu_hbm, s2_hbm, d2_hbm, eo_hbm, nsp, srcv, dgv, rs, rd, eo,
          ss0, ss1, sd0, sd1, sw0, sw1):
        cid = lax.axis_index("c")
        sid = lax.axis_index("s")
        wid = sid * _NC + cid
        row0 = wid * nit
        e0 = row0 * _CH
        @pl.when(sid < _NS - 1)
        def _():
            pltpu.sync_copy(nu_hbm.at[pl.ds(sid * rpt, rpt)],
                            nsp.at[pl.ds(sid * rpt, rpt)])

        @pl.when(sid == _NS - 1)
        def _():
            pltpu.sync_copy(nu_hbm.at[pl.ds(rpt * (_NS - 1), last)],
                            nsp.at[pl.ds(rpt * (_NS - 1), last)])

        pltpu.sync_copy(s2_hbm.at[pl.ds(row0, nit)], srcv)
        pltpu.sync_copy(d2_hbm.at[pl.ds(row0, nit)], dgv)
        plsc.subcore_barrier()
        ssem = (ss0, ss1)
        dsem = (sd0, sd1)
        wsem = (sw0, sw1)
        for bs in range(2):
            pltpu.async_copy(nsp.at[srcv.at[bs]], rs.at[bs], ssem[bs])
            pltpu.async_copy(nsp.at[dgv.at[bs]], rd.at[bs], dsem[bs])

        def pair(g, carry):
            for bs in range(2):
                c = 2 * g + bs
                pltpu.make_async_copy(nsp.at[srcv.at[c]], rs.at[bs],
                                      ssem[bs]).wait()
                pltpu.make_async_copy(nsp.at[dgv.at[c]], rd.at[bs],
                                      dsem[bs]).wait()

                @pl.when(c >= 2)
                def _():
                    pltpu.make_async_copy(
                        eo.at[bs], eo_hbm.at[pl.ds(e0 + (c - 2) * _CH, _CH)],
                        wsem[bs]).wait()

                def rowf(r, c2):
                    for j in range(hw):
                        sl = pl.ds(j * 32, 32)
                        s2 = rs[bs, r, sl] + rd[bs, r, sl]
                        lo, hi = plsc.unpack(
                            s2, format=plsc.PackFormat.INTERLEAVED,
                            preferred_element_type=F32)
                        eo[bs, r, pl.ds(j * 32, 16)] = lo * 0.5
                        eo[bs, r, pl.ds(j * 32 + 16, 16)] = hi * 0.5
                    return c2

                lax.fori_loop(0, _CH, rowf, 0, unroll=4)
                pltpu.async_copy(eo.at[bs],
                                 eo_hbm.at[pl.ds(e0 + c * _CH, _CH)], wsem[bs])

                @pl.when(c + 2 < nit)
                def _():
                    pltpu.async_copy(nsp.at[srcv.at[c + 2]], rs.at[bs],
                                     ssem[bs])
                    pltpu.async_copy(nsp.at[dgv.at[c + 2]], rd.at[bs],
                                     dsem[bs])
            return carry

        lax.fori_loop(0, nit // 2, pair, 0)
        for bs in range(2):
            c = nit - 2 + bs
            pltpu.make_async_copy(eo.at[bs],
                                  eo_hbm.at[pl.ds(e0 + c * _CH, _CH)],
                                  wsem[bs]).wait()

    return k(nupd, src2d, dstg2d)


def _final_tc(p0, p1, h, n_blk=1000):
    n, hx = p0.shape

    def body(p0_ref, p1_ref, out_ref):
        s = p0_ref[:, :h] + p1_ref[:, :h]
        d = p0_ref[:, h:h + 16] + p1_ref[:, h:h + 16]
        deg = jnp.maximum(d[:, 0:1], 1.0)
        out_ref[...] = s / deg

    return pl.pallas_call(
        body,
        grid=(n // n_blk,),
        in_specs=[
            pl.BlockSpec((n_blk, hx), lambda i: (i, 0)),
            pl.BlockSpec((n_blk, hx), lambda i: (i, 0)),
        ],
        out_specs=pl.BlockSpec((n_blk, h), lambda i: (i, 0)),
        out_shape=jax.ShapeDtypeStruct((n, h), F32),
    )(p0, p1)


def kernel(node_attr, edge_idx, edge_attr, We1, be1, We2, be2,
           Wn1, bn1, Wn2, bn2):
    n, nd = node_attr.shape
    e, ed = edge_attr.shape
    h = We2.shape[1]
    src = edge_idx[0]
    dst = edge_idx[1]

    grain = _NW * _CH * 2
    e_pad = -(-e // grain) * grain
    pad = e_pad - e
    i32 = jnp.int32
    src_p = jnp.concatenate([src, jnp.zeros((pad,), i32)]).reshape(-1, _CH)
    dst_g = jnp.concatenate([dst, jnp.zeros((pad,), i32)]).reshape(-1, _CH)
    dst_s = jnp.concatenate([dst, jnp.full((pad,), n, i32)]).reshape(-1, _CH)

    a, b = _prep_tc(node_attr, We1[:nd], We1[nd:2 * nd])
    zrows = jnp.zeros((_rpt(n + 1), h), F32)

    half = e // 2
    sgrain = _NW * _CH * 2
    s_pad = -(-half // sgrain) * sgrain
    spad = s_pad - half
    aggp = None
    for lo in (0, half):
        src_s = jnp.concatenate([src[lo:lo + half],
                                 jnp.zeros((spad,), i32)]).reshape(-1, _CH)
        dstg_s = jnp.concatenate([dst[lo:lo + half],
                                  jnp.zeros((spad,), i32)]).reshape(-1, _CH)
        dsts_s = jnp.concatenate([dst[lo:lo + half],
                                  jnp.full((spad,), n, i32)]).reshape(-1, _CH)
        hpre = _sc_gather_sum(a, b, src_s, dstg_s)
        e_upd = _edge_mlp_tc(hpre, edge_attr, We1[2 * nd:],
                             be1.reshape(1, -1), We2, be2.reshape(1, -1),
                             nreal=half, ea_off=lo)
        p = _sc_scatter_add(e_upd, dsts_s, zrows, n)
        aggp = p if aggp is None else aggp + p
    n_upd, nupd_bf = _node_mlp_tc(node_attr, aggp[0, :n], aggp[1, :n],
                                  Wn1[:nd], Wn1[nd:], bn1.reshape(1, -1),
                                  Wn2, bn2.reshape(1, -1))
    hx = h + 16
    ext = jnp.concatenate(
        [n_upd, jnp.ones((n, 1), F32), jnp.zeros((n, hx - h - 1), F32)],
        axis=1)
    zrows_x = jnp.zeros((_rpt(n + 1), hx), F32)
    nbrp = _sc_nbr_scatter(ext, src_p, dst_s, zrows_x, n)
    nupd_perm = (nupd_bf.reshape(n, h // 32, 2, 16)
                 .transpose(0, 1, 3, 2).reshape(n, h))
    edge_out = _sc_edge_avg(nupd_perm, src_p, dst_g)
    node_out = _final_tc(nbrp[0, :n], nbrp[1, :n], h)
    return (node_out, edge_out[:e])
